# Initial kernel scaffold; baseline (speedup 1.0000x reference)
import functools

import jax
import jax.numpy as jnp
from jax import lax
from jax.experimental import pallas as pl
from jax.experimental.pallas import tpu as pltpu
from jax.experimental.pallas import tpu_sc as plsc

N = 10000
NP = 10240
E = 320000
D = 128
DEGW = 16

NC = 2
NS = 16
NW = NC * NS
EW = E // NW
CHUNK = 128
NFULL = EW // CHUNK
TAIL = EW - NFULL * CHUNK
SEG = NP // NS

BLK = 1280
GRID = NP // BLK

_mesh = plsc.VectorSubcoreMesh(core_axis_name="c", subcore_axis_name="s")


@functools.partial(
    pl.kernel,
    out_type=jax.ShapeDtypeStruct((NC, NP, DEGW), jnp.float32),
    mesh=_mesh,
    scratch_types=[
        pltpu.VMEM((CHUNK,), jnp.int32),
        pltpu.VMEM((TAIL,), jnp.int32),
        pltpu.VMEM((CHUNK, DEGW), jnp.float32),
        pltpu.VMEM((TAIL, DEGW), jnp.float32),
        pltpu.VMEM_SHARED((NP, DEGW), jnp.float32),
    ],
)
def _deg_kernel(dst_hbm, ones_hbm, zeros_hbm, out_hbm,
                idx_v, idxt_v, ones_v, onest_v, acc_sh):
    c = lax.axis_index("c")
    s = lax.axis_index("s")
    wid = c * NS + s
    base = wid * EW

    pltpu.sync_copy(zeros_hbm, acc_sh.at[pl.ds(s * SEG, SEG)])
    pltpu.sync_copy(ones_hbm, ones_v)
    pltpu.sync_copy(ones_hbm.at[pl.ds(0, TAIL)], onest_v)
    plsc.subcore_barrier()

    def body(i, carry):
        off = pl.multiple_of(base + i * CHUNK, 8)
        pltpu.sync_copy(dst_hbm.at[pl.ds(off, CHUNK)], idx_v)
        pltpu.sync_copy(ones_v, acc_sh.at[idx_v], add=True)
        return carry

    lax.fori_loop(0, NFULL, body, 0)
    offt = pl.multiple_of(base + NFULL * CHUNK, 8)
    pltpu.sync_copy(dst_hbm.at[pl.ds(offt, TAIL)], idxt_v)
    pltpu.sync_copy(onest_v, acc_sh.at[idxt_v], add=True)

    plsc.subcore_barrier()
    pltpu.sync_copy(acc_sh.at[pl.ds(s * SEG, SEG)],
                    out_hbm.at[c, pl.ds(s * SEG, SEG)])


@functools.partial(
    pl.kernel,
    out_type=jax.ShapeDtypeStruct((NC, NP, D), jnp.float32),
    mesh=_mesh,
    scratch_types=[
        pltpu.VMEM((CHUNK,), jnp.int32),
        pltpu.VMEM((CHUNK,), jnp.int32),
        pltpu.VMEM((TAIL,), jnp.int32),
        pltpu.VMEM((TAIL,), jnp.int32),
        pltpu.VMEM((CHUNK, D), jnp.float32),
        pltpu.VMEM((TAIL, D), jnp.float32),
        pltpu.SemaphoreType.DMA,
        pltpu.VMEM_SHARED((NP, D), jnp.float32),
    ],
)
def _msg_kernel(g_hbm, src_hbm, dst_hbm, zeros_hbm, out_hbm,
                si_v, di_v, sit_v, dit_v, rows_v, rowst_v, sem, acc_sh):
    c = lax.axis_index("c")
    s = lax.axis_index("s")
    wid = c * NS + s
    base = wid * EW

    pltpu.sync_copy(zeros_hbm, acc_sh.at[pl.ds(s * SEG, SEG)])
    plsc.subcore_barrier()

    def body(i, carry):
        off = pl.multiple_of(base + i * CHUNK, 8)
        pltpu.sync_copy(src_hbm.at[pl.ds(off, CHUNK)], si_v)
        pltpu.sync_copy(dst_hbm.at[pl.ds(off, CHUNK)], di_v)
        pltpu.async_copy(g_hbm.at[si_v], rows_v, sem).wait()
        pltpu.sync_copy(rows_v, acc_sh.at[di_v], add=True)
        return carry

    lax.fori_loop(0, NFULL, body, 0)
    offt = pl.multiple_of(base + NFULL * CHUNK, 8)
    pltpu.sync_copy(src_hbm.at[pl.ds(offt, TAIL)], sit_v)
    pltpu.sync_copy(dst_hbm.at[pl.ds(offt, TAIL)], dit_v)
    pltpu.async_copy(g_hbm.at[sit_v], rowst_v, sem).wait()
    pltpu.sync_copy(rowst_v, acc_sh.at[dit_v], add=True)

    plsc.subcore_barrier()
    pltpu.sync_copy(acc_sh.at[pl.ds(s * SEG, SEG)],
                    out_hbm.at[c, pl.ds(s * SEG, SEG)])


def _dinv_block(deg_blk):
    d = (deg_blk[0] + deg_blk[1])[:, 0:1]
    return jnp.where(d > 0.0, lax.rsqrt(d), 0.0)


def _k0_body(deg_ref, x_ref, we_ref, w0_ref, h_ref, g_ref):
    h = jnp.dot(x_ref[...], we_ref[...], preferred_element_type=jnp.float32)
    dinv = _dinv_block(deg_ref[...])
    g_ref[...] = jnp.dot(h, w0_ref[...],
                         preferred_element_type=jnp.float32) * dinv
    h_ref[...] = h


def _kmid_body(deg_ref, p_ref, hp_ref, b_ref, w_ref, h_ref, g_ref):
    dinv = _dinv_block(deg_ref[...])
    p = p_ref[0] + p_ref[1]
    conv = p * dinv + b_ref[...]
    h = hp_ref[...] + jnp.maximum(conv, 0.0)
    g_ref[...] = jnp.dot(h, w_ref[...],
                         preferred_element_type=jnp.float32) * dinv
    h_ref[...] = h


def _kfin_body(deg_ref, p_ref, hp_ref, b_ref, h_ref):
    dinv = _dinv_block(deg_ref[...])
    p = p_ref[0] + p_ref[1]
    conv = p * dinv + b_ref[...]
    h_ref[...] = hp_ref[...] + jnp.maximum(conv, 0.0)


_spec_deg = pl.BlockSpec((NC, BLK, DEGW), lambda i: (0, i, 0))
_spec_node = pl.BlockSpec((BLK, D), lambda i: (i, 0))
_spec_w = pl.BlockSpec((D, D), lambda i: (0, 0))
_spec_p = pl.BlockSpec((NC, BLK, D), lambda i: (0, i, 0))
_spec_b = pl.BlockSpec((1, D), lambda i: (0, 0))

_node_out = jax.ShapeDtypeStruct((NP, D), jnp.float32)

_k0 = pl.pallas_call(
    _k0_body,
    grid=(GRID,),
    in_specs=[_spec_deg, _spec_node, _spec_w, _spec_w],
    out_specs=[_spec_node, _spec_node],
    out_shape=[_node_out, _node_out],
)

_kmid = pl.pallas_call(
    _kmid_body,
    grid=(GRID,),
    in_specs=[_spec_deg, _spec_p, _spec_node, _spec_b, _spec_w],
    out_specs=[_spec_node, _spec_node],
    out_shape=[_node_out, _node_out],
)

_kfin = pl.pallas_call(
    _kfin_body,
    grid=(GRID,),
    in_specs=[_spec_deg, _spec_p, _spec_node, _spec_b],
    out_specs=[_spec_node],
    out_shape=[_node_out],
)


def kernel(x, edge_index, W_enc, W0, b0, W1, b1, W2, b2):
    src = edge_index[0]
    dst = edge_index[1]
    xp = jnp.pad(x, ((0, NP - N), (0, 0)))
    ones_deg = jnp.ones((CHUNK, DEGW), jnp.float32)
    zeros_deg = jnp.zeros((SEG, DEGW), jnp.float32)
    zeros_msg = jnp.zeros((SEG, D), jnp.float32)

    deg = _deg_kernel(dst, ones_deg, zeros_deg)
    h, g = _k0(deg, xp, W_enc, W0)
    p = _msg_kernel(g, src, dst, zeros_msg)
    h, g = _kmid(deg, p, h, b0.reshape(1, D), W1)
    p = _msg_kernel(g, src, dst, zeros_msg)
    h, g = _kmid(deg, p, h, b1.reshape(1, D), W2)
    p = _msg_kernel(g, src, dst, zeros_msg)
    h = _kfin(deg, p, h, b2.reshape(1, D))
    return h[:N]


# speedup vs baseline: 10.4924x; 10.4924x over previous
import functools

import jax
import jax.numpy as jnp
from jax import lax
from jax.experimental import pallas as pl
from jax.experimental.pallas import tpu as pltpu
from jax.experimental.pallas import tpu_sc as plsc

N = 10000
NP = 10240
E = 320000
D = 128
DEGW = 128

NC = 2
NS = 16
NW = NC * NS
EW = E // NW
CHUNK = 128
NFULL = EW // CHUNK
TAIL = EW - NFULL * CHUNK
SEG = NP // NS

BLK = 1280
GRID = NP // BLK

_mesh = plsc.VectorSubcoreMesh(core_axis_name="c", subcore_axis_name="s")


_deg_scratch = [
    pltpu.VMEM((CHUNK,), jnp.int32),
    pltpu.VMEM((TAIL,), jnp.int32),
    pltpu.VMEM((CHUNK, DEGW), jnp.float32),
    pltpu.VMEM((TAIL, DEGW), jnp.float32),
    pltpu.VMEM_SHARED((NP, DEGW), jnp.float32),
]


def _deg_body(dst_hbm, ones_hbm, zeros_hbm, out_hbm,
                idx_v, idxt_v, ones_v, onest_v, acc_sh):
    c = lax.axis_index("c")
    s = lax.axis_index("s")
    wid = c * NS + s
    base = wid * EW

    pltpu.sync_copy(zeros_hbm, acc_sh.at[pl.ds(s * SEG, SEG)])
    pltpu.sync_copy(ones_hbm, ones_v)
    pltpu.sync_copy(ones_hbm.at[pl.ds(0, TAIL)], onest_v)
    plsc.subcore_barrier()

    def body(i, carry):
        off = pl.multiple_of(base + i * CHUNK, 8)
        pltpu.sync_copy(dst_hbm.at[pl.ds(off, CHUNK)], idx_v)
        pltpu.sync_copy(ones_v, acc_sh.at[idx_v], add=True)
        return carry

    lax.fori_loop(0, NFULL, body, 0)
    offt = pl.multiple_of(base + NFULL * CHUNK, 8)
    pltpu.sync_copy(dst_hbm.at[pl.ds(offt, TAIL)], idxt_v)
    pltpu.sync_copy(onest_v, acc_sh.at[idxt_v], add=True)

    plsc.subcore_barrier()
    pltpu.sync_copy(acc_sh.at[pl.ds(s * SEG, SEG)],
                    out_hbm.at[c, pl.ds(s * SEG, SEG)])


_msg_scratch = [
    pltpu.VMEM((CHUNK,), jnp.int32),
    pltpu.VMEM((CHUNK,), jnp.int32),
    pltpu.VMEM((TAIL,), jnp.int32),
    pltpu.VMEM((TAIL,), jnp.int32),
    pltpu.VMEM((CHUNK, D), jnp.float32),
    pltpu.VMEM((TAIL, D), jnp.float32),
    pltpu.SemaphoreType.DMA,
    pltpu.VMEM_SHARED((NP, D), jnp.float32),
]


def _msg_body(g_hbm, src_hbm, dst_hbm, zeros_hbm, out_hbm,
                si_v, di_v, sit_v, dit_v, rows_v, rowst_v, sem, acc_sh):
    c = lax.axis_index("c")
    s = lax.axis_index("s")
    wid = c * NS + s
    base = wid * EW

    pltpu.sync_copy(zeros_hbm, acc_sh.at[pl.ds(s * SEG, SEG)])
    plsc.subcore_barrier()

    def body(i, carry):
        off = pl.multiple_of(base + i * CHUNK, 8)
        pltpu.sync_copy(src_hbm.at[pl.ds(off, CHUNK)], si_v)
        pltpu.sync_copy(dst_hbm.at[pl.ds(off, CHUNK)], di_v)
        pltpu.async_copy(g_hbm.at[si_v], rows_v, sem).wait()
        pltpu.sync_copy(rows_v, acc_sh.at[di_v], add=True)
        return carry

    lax.fori_loop(0, NFULL, body, 0)
    offt = pl.multiple_of(base + NFULL * CHUNK, 8)
    pltpu.sync_copy(src_hbm.at[pl.ds(offt, TAIL)], sit_v)
    pltpu.sync_copy(dst_hbm.at[pl.ds(offt, TAIL)], dit_v)
    pltpu.async_copy(g_hbm.at[sit_v], rowst_v, sem).wait()
    pltpu.sync_copy(rowst_v, acc_sh.at[dit_v], add=True)

    plsc.subcore_barrier()
    pltpu.sync_copy(acc_sh.at[pl.ds(s * SEG, SEG)],
                    out_hbm.at[c, pl.ds(s * SEG, SEG)])


_deg_kernel = pl.kernel(
    _deg_body,
    out_type=jax.ShapeDtypeStruct((NC, NP, DEGW), jnp.float32),
    mesh=_mesh,
    scratch_types=_deg_scratch,
)

_msg_kernel = pl.kernel(
    _msg_body,
    out_type=jax.ShapeDtypeStruct((NC, NP, D), jnp.float32),
    mesh=_mesh,
    scratch_types=_msg_scratch,
)


def _dinv_block(deg_blk):
    d = (deg_blk[0] + deg_blk[1])[:, 0:1]
    return jnp.where(d > 0.0, lax.rsqrt(d), 0.0)


def _k0_body(deg_ref, x_ref, we_ref, w0_ref, h_ref, g_ref):
    h = jnp.dot(x_ref[...], we_ref[...], preferred_element_type=jnp.float32)
    dinv = _dinv_block(deg_ref[...])
    g_ref[...] = jnp.dot(h, w0_ref[...],
                         preferred_element_type=jnp.float32) * dinv
    h_ref[...] = h


def _kmid_body(deg_ref, p_ref, hp_ref, b_ref, w_ref, h_ref, g_ref):
    dinv = _dinv_block(deg_ref[...])
    p = p_ref[0] + p_ref[1]
    conv = p * dinv + b_ref[...]
    h = hp_ref[...] + jnp.maximum(conv, 0.0)
    g_ref[...] = jnp.dot(h, w_ref[...],
                         preferred_element_type=jnp.float32) * dinv
    h_ref[...] = h


def _kfin_body(deg_ref, p_ref, hp_ref, b_ref, h_ref):
    dinv = _dinv_block(deg_ref[...])
    p = p_ref[0] + p_ref[1]
    conv = p * dinv + b_ref[...]
    h_ref[...] = hp_ref[...] + jnp.maximum(conv, 0.0)


_spec_deg = pl.BlockSpec((NC, BLK, DEGW), lambda i: (0, i, 0))
_spec_node = pl.BlockSpec((BLK, D), lambda i: (i, 0))
_spec_w = pl.BlockSpec((D, D), lambda i: (0, 0))
_spec_p = pl.BlockSpec((NC, BLK, D), lambda i: (0, i, 0))
_spec_b = pl.BlockSpec((1, D), lambda i: (0, 0))

_node_out = jax.ShapeDtypeStruct((NP, D), jnp.float32)

_k0 = pl.pallas_call(
    _k0_body,
    grid=(GRID,),
    in_specs=[_spec_deg, _spec_node, _spec_w, _spec_w],
    out_specs=[_spec_node, _spec_node],
    out_shape=[_node_out, _node_out],
)

_kmid = pl.pallas_call(
    _kmid_body,
    grid=(GRID,),
    in_specs=[_spec_deg, _spec_p, _spec_node, _spec_b, _spec_w],
    out_specs=[_spec_node, _spec_node],
    out_shape=[_node_out, _node_out],
)

_kfin = pl.pallas_call(
    _kfin_body,
    grid=(GRID,),
    in_specs=[_spec_deg, _spec_p, _spec_node, _spec_b],
    out_specs=_spec_node,
    out_shape=_node_out,
)


def kernel(x, edge_index, W_enc, W0, b0, W1, b1, W2, b2):
    src = edge_index[0]
    dst = edge_index[1]
    xp = jnp.pad(x, ((0, NP - N), (0, 0)))
    ones_deg = jnp.ones((CHUNK, DEGW), jnp.float32)
    zeros_deg = jnp.zeros((SEG, DEGW), jnp.float32)
    zeros_msg = jnp.zeros((SEG, D), jnp.float32)

    deg = _deg_kernel(dst, ones_deg, zeros_deg)
    h, g = _k0(deg, xp, W_enc, W0)
    p = _msg_kernel(g, src, dst, zeros_msg)
    h, g = _kmid(deg, p, h, b0.reshape(1, D), W1)
    p = _msg_kernel(g, src, dst, zeros_msg)
    h, g = _kmid(deg, p, h, b1.reshape(1, D), W2)
    p = _msg_kernel(g, src, dst, zeros_msg)
    h = _kfin(deg, p, h, b2.reshape(1, D))
    return h[:N]


# speedup vs baseline: 19.8886x; 1.8955x over previous
import jax
import jax.numpy as jnp
from jax import lax
from jax.experimental import pallas as pl
from jax.experimental.pallas import tpu as pltpu
from jax.experimental.pallas import tpu_sc as plsc

N = 10000
NP = 10240
E = 320000
D = 128
DEGW = 128

NC = 2
NS = 16
NW = NC * NS
CHUNK = 128
NCHUNK = E // CHUNK
NFULL = NCHUNK // NW
NEXTRA = NCHUNK - NFULL * NW
SEG = NP // NS

BLK = 1280
GRID = NP // BLK

_mesh = plsc.VectorSubcoreMesh(core_axis_name="c", subcore_axis_name="s")


_deg_scratch = [
    pltpu.VMEM((CHUNK,), jnp.int32),
    pltpu.VMEM((CHUNK,), jnp.int32),
    pltpu.VMEM((CHUNK, DEGW), jnp.float32),
    pltpu.SemaphoreType.DMA,
    pltpu.SemaphoreType.DMA,
    pltpu.VMEM_SHARED((NP, DEGW), jnp.float32),
]


def _deg_body(dst_hbm, ones_hbm, zeros_hbm, out_hbm,
              di0, di1, ones_v, smi0, smi1, acc_sh):
    c = lax.axis_index("c")
    s = lax.axis_index("s")
    wid = c * NS + s
    di = (di0, di1)
    smi = (smi0, smi1)

    pltpu.sync_copy(zeros_hbm, acc_sh.at[pl.ds(s * SEG, SEG)])
    pltpu.sync_copy(ones_hbm, ones_v)
    plsc.subcore_barrier()

    def issue_idx(k, slot):
        off = pl.multiple_of((wid + NW * k) * CHUNK, CHUNK)
        pltpu.async_copy(dst_hbm.at[pl.ds(off, CHUNK)], di[slot], smi[slot])

    def wait_idx(slot):
        pltpu.make_async_copy(dst_hbm.at[pl.ds(0, CHUNK)], di[slot],
                              smi[slot]).wait()

    def scatter(slot):
        pltpu.sync_copy(ones_v, acc_sh.at[di[slot]], add=True)

    issue_idx(0, 0)
    issue_idx(1, 1)

    def body(j, carry):
        for t in range(2):
            k = 2 * j + t
            wait_idx(t)
            scatter(t)
            issue_idx(k + 2, t)
        return carry

    lax.fori_loop(0, NFULL // 2 - 1, body, 0)
    for k in (NFULL - 2, NFULL - 1):
        wait_idx(k % 2)
        scatter(k % 2)

    @pl.when(wid < NEXTRA)
    def _():
        offx = pl.multiple_of((wid + NW * NFULL) * CHUNK, CHUNK)
        pltpu.sync_copy(dst_hbm.at[pl.ds(offx, CHUNK)], di0)
        scatter(0)

    plsc.subcore_barrier()
    pltpu.sync_copy(acc_sh.at[pl.ds(s * SEG, SEG)],
                    out_hbm.at[c, pl.ds(s * SEG, SEG)])


_msg_scratch = [
    pltpu.VMEM((CHUNK,), jnp.int32),
    pltpu.VMEM((CHUNK,), jnp.int32),
    pltpu.VMEM((CHUNK,), jnp.int32),
    pltpu.VMEM((CHUNK,), jnp.int32),
    pltpu.VMEM((CHUNK,), jnp.int32),
    pltpu.VMEM((CHUNK,), jnp.int32),
    pltpu.VMEM((CHUNK,), jnp.int32),
    pltpu.VMEM((CHUNK,), jnp.int32),
    pltpu.VMEM((CHUNK, D), jnp.float32),
    pltpu.VMEM((CHUNK, D), jnp.float32),
    pltpu.SemaphoreType.DMA,
    pltpu.SemaphoreType.DMA,
    pltpu.SemaphoreType.DMA,
    pltpu.SemaphoreType.DMA,
    pltpu.SemaphoreType.DMA,
    pltpu.SemaphoreType.DMA,
    pltpu.VMEM_SHARED((NP, D), jnp.float32),
]


def _msg_body(g_hbm, src_hbm, dst_hbm, zeros_hbm, out_hbm,
              si0, si1, si2, si3, di0, di1, di2, di3, rows0, rows1,
              smi0, smi1, smi2, smi3, smg0, smg1, acc_sh):
    c = lax.axis_index("c")
    s = lax.axis_index("s")
    wid = c * NS + s
    si = (si0, si1, si2, si3)
    di = (di0, di1, di2, di3)
    smi = (smi0, smi1, smi2, smi3)
    rows = (rows0, rows1)
    smg = (smg0, smg1)

    pltpu.sync_copy(zeros_hbm, acc_sh.at[pl.ds(s * SEG, SEG)])
    plsc.subcore_barrier()

    def issue_idx(k, slot):
        off = pl.multiple_of((wid + NW * k) * CHUNK, CHUNK)
        pltpu.async_copy(src_hbm.at[pl.ds(off, CHUNK)], si[slot], smi[slot])
        pltpu.async_copy(dst_hbm.at[pl.ds(off, CHUNK)], di[slot], smi[slot])

    def wait_idx(slot):
        pltpu.make_async_copy(src_hbm.at[pl.ds(0, CHUNK)], si[slot],
                              smi[slot]).wait()
        pltpu.make_async_copy(src_hbm.at[pl.ds(0, CHUNK)], di[slot],
                              smi[slot]).wait()

    def issue_gather(islot, rslot):
        pltpu.async_copy(g_hbm.at[si[islot]], rows[rslot], smg[rslot])

    def wait_gather(rslot):
        pltpu.make_async_copy(g_hbm.at[pl.ds(0, CHUNK)], rows[rslot],
                              smg[rslot]).wait()

    def scatter(islot, rslot):
        pltpu.sync_copy(rows[rslot], acc_sh.at[di[islot]], add=True)

    for k in range(4):
        issue_idx(k, k)
    wait_idx(0)
    issue_gather(0, 0)

    def step(k, ks, issue_next_gather, issue_next_idx):
        if issue_next_gather:
            wait_idx((ks + 1) % 4)
            issue_gather((ks + 1) % 4, (ks + 1) % 2)
        wait_gather(ks % 2)
        scatter(ks % 4, ks % 2)
        if issue_next_idx:
            issue_idx(k + 4, ks % 4)

    def body(j, carry):
        for t in range(4):
            step(4 * j + t, t, True, True)
        return carry

    lax.fori_loop(0, NFULL // 4 - 2, body, 0)
    for k in range(4 * (NFULL // 4 - 2), NFULL):
        step(k, k % 4, k + 1 < NFULL, k + 4 < NFULL)

    @pl.when(wid < NEXTRA)
    def _():
        offx = pl.multiple_of((wid + NW * NFULL) * CHUNK, CHUNK)
        pltpu.sync_copy(src_hbm.at[pl.ds(offx, CHUNK)], si0)
        pltpu.sync_copy(dst_hbm.at[pl.ds(offx, CHUNK)], di0)
        pltpu.async_copy(g_hbm.at[si0], rows0, smg0).wait()
        pltpu.sync_copy(rows0, acc_sh.at[di0], add=True)

    plsc.subcore_barrier()
    pltpu.sync_copy(acc_sh.at[pl.ds(s * SEG, SEG)],
                    out_hbm.at[c, pl.ds(s * SEG, SEG)])


_deg_kernel = pl.kernel(
    _deg_body,
    out_type=jax.ShapeDtypeStruct((NC, NP, DEGW), jnp.float32),
    mesh=_mesh,
    scratch_types=_deg_scratch,
)

_msg_kernel = pl.kernel(
    _msg_body,
    out_type=jax.ShapeDtypeStruct((NC, NP, D), jnp.float32),
    mesh=_mesh,
    scratch_types=_msg_scratch,
)


def _dinv_block(deg_blk):
    d = (deg_blk[0] + deg_blk[1])[:, 0:1]
    return jnp.where(d > 0.0, lax.rsqrt(d), 0.0)


def _k0_body(deg_ref, x_ref, we_ref, w0_ref, h_ref, g_ref):
    h = jnp.dot(x_ref[...], we_ref[...], preferred_element_type=jnp.float32)
    dinv = _dinv_block(deg_ref[...])
    g_ref[...] = jnp.dot(h, w0_ref[...],
                         preferred_element_type=jnp.float32) * dinv
    h_ref[...] = h


def _kmid_body(deg_ref, p_ref, hp_ref, b_ref, w_ref, h_ref, g_ref):
    dinv = _dinv_block(deg_ref[...])
    p = p_ref[0] + p_ref[1]
    conv = p * dinv + b_ref[...]
    h = hp_ref[...] + jnp.maximum(conv, 0.0)
    g_ref[...] = jnp.dot(h, w_ref[...],
                         preferred_element_type=jnp.float32) * dinv
    h_ref[...] = h


def _kfin_body(deg_ref, p_ref, hp_ref, b_ref, h_ref):
    dinv = _dinv_block(deg_ref[...])
    p = p_ref[0] + p_ref[1]
    conv = p * dinv + b_ref[...]
    h_ref[...] = hp_ref[...] + jnp.maximum(conv, 0.0)


_spec_deg = pl.BlockSpec((NC, BLK, DEGW), lambda i: (0, i, 0))
_spec_node = pl.BlockSpec((BLK, D), lambda i: (i, 0))
_spec_w = pl.BlockSpec((D, D), lambda i: (0, 0))
_spec_p = pl.BlockSpec((NC, BLK, D), lambda i: (0, i, 0))
_spec_b = pl.BlockSpec((1, D), lambda i: (0, 0))

_node_out = jax.ShapeDtypeStruct((NP, D), jnp.float32)

_k0 = pl.pallas_call(
    _k0_body,
    grid=(GRID,),
    in_specs=[_spec_deg, _spec_node, _spec_w, _spec_w],
    out_specs=[_spec_node, _spec_node],
    out_shape=[_node_out, _node_out],
)

_kmid = pl.pallas_call(
    _kmid_body,
    grid=(GRID,),
    in_specs=[_spec_deg, _spec_p, _spec_node, _spec_b, _spec_w],
    out_specs=[_spec_node, _spec_node],
    out_shape=[_node_out, _node_out],
)

_kfin = pl.pallas_call(
    _kfin_body,
    grid=(GRID,),
    in_specs=[_spec_deg, _spec_p, _spec_node, _spec_b],
    out_specs=_spec_node,
    out_shape=_node_out,
)


def kernel(x, edge_index, W_enc, W0, b0, W1, b1, W2, b2):
    src = edge_index[0]
    dst = edge_index[1]
    xp = jnp.pad(x, ((0, NP - N), (0, 0)))
    ones_deg = jnp.ones((CHUNK, DEGW), jnp.float32)
    zeros_deg = jnp.zeros((SEG, DEGW), jnp.float32)
    zeros_msg = jnp.zeros((SEG, D), jnp.float32)

    deg = _deg_kernel(dst, ones_deg, zeros_deg)
    h, g = _k0(deg, xp, W_enc, W0)
    p = _msg_kernel(g, src, dst, zeros_msg)
    h, g = _kmid(deg, p, h, b0.reshape(1, D), W1)
    p = _msg_kernel(g, src, dst, zeros_msg)
    h, g = _kmid(deg, p, h, b1.reshape(1, D), W2)
    p = _msg_kernel(g, src, dst, zeros_msg)
    h = _kfin(deg, p, h, b2.reshape(1, D))
    return h[:N]
